# Initial kernel scaffold; baseline (speedup 1.0000x reference)
import jax
import jax.numpy as jnp
from jax.experimental import pallas as pl


def kernel(x, edge_index, W_emb, b_emb, W_g, b_g, W_ih, W_hh, b_ih, b_hh, W_out, b_out):
    raise NotImplementedError("write your pallas kernel here")


# speedup vs baseline: 5.9520x; 5.9520x over previous
import functools

import jax
import jax.numpy as jnp
from jax import lax
from jax.experimental import pallas as pl
from jax.experimental.pallas import tpu as pltpu
from jax.experimental.pallas import tpu_sc as plsc

N_NODES = 10000
N_EDGES = 320000
HID = 128
N_STEPS = 3

ROW_BLK = 1000
GRID = N_NODES // ROW_BLK

NC = 2
NS = 16
NW = NC * NS
CHUNK = 128
N_CHUNKS = N_EDGES // CHUNK
CHUNKS_FULL = N_CHUNKS // NW
CHUNKS_REM = N_CHUNKS % NW
N_PAD = 10240
ROWS_PER_TILE = N_PAD // NS

def _dot(a, b):
    return jnp.dot(a.astype(jnp.bfloat16), b.astype(jnp.bfloat16),
                   preferred_element_type=jnp.float32)


def _embed_body(x_ref, wembt, bemb, wgt, bg, whht, bhh, h_ref, m_ref, gh_ref):
    x = x_ref[...]
    h = _dot(x, wembt[...]) + bemb[...]
    h_ref[...] = h
    m_ref[...] = _dot(h, wgt[...]) + bg[...]
    gh_ref[...] = _dot(h, whht[...]) + bhh[...]


def _gru_core(a0_ref, a1_ref, h_ref, gh_ref, wiht, bih):
    a = a0_ref[0] + a1_ref[0]
    gi = _dot(a, wiht[...]) + bih[...]
    gh = gh_ref[...]
    h = h_ref[...]
    r = jax.nn.sigmoid(gi[:, :HID] + gh[:, :HID])
    z = jax.nn.sigmoid(gi[:, HID:2 * HID] + gh[:, HID:2 * HID])
    n = jnp.tanh(gi[:, 2 * HID:] + r * gh[:, 2 * HID:])
    return (1.0 - z) * n + z * h


def _gru_mid_body(a0_ref, a1_ref, h_ref, gh_ref, wiht, bih, wgt, bg, whht, bhh,
                  ho_ref, mo_ref, gho_ref):
    hn = _gru_core(a0_ref, a1_ref, h_ref, gh_ref, wiht, bih)
    ho_ref[...] = hn
    mo_ref[...] = _dot(hn, wgt[...]) + bg[...]
    gho_ref[...] = _dot(hn, whht[...]) + bhh[...]


def _gru_last_body(a0_ref, a1_ref, h_ref, gh_ref, wiht, bih, woutt, bout,
                   out_ref):
    hn = _gru_core(a0_ref, a1_ref, h_ref, gh_ref, wiht, bih)
    out_ref[...] = jnp.tanh(_dot(hn, woutt[...]) + bout[...])


def _row_spec(cols):
    return pl.BlockSpec((ROW_BLK, cols), lambda i: (i, 0))


def _full_spec(r, c):
    return pl.BlockSpec((r, c), lambda i: (0, 0))


def _part_spec(which):
    return pl.BlockSpec((1, ROW_BLK, HID), lambda i, w=which: (w, i, 0))


_f32 = jnp.float32


def _embed_call(x, wembt, bemb, wgt, bg, whht, bhh):
    return pl.pallas_call(
        _embed_body,
        grid=(GRID,),
        in_specs=[
            _row_spec(HID),
            _full_spec(HID, HID), _full_spec(1, HID),
            _full_spec(HID, HID), _full_spec(1, HID),
            _full_spec(HID, 3 * HID), _full_spec(1, 3 * HID),
        ],
        out_specs=[_row_spec(HID), _row_spec(HID), _row_spec(3 * HID)],
        out_shape=[
            jax.ShapeDtypeStruct((N_NODES, HID), _f32),
            jax.ShapeDtypeStruct((N_NODES, HID), _f32),
            jax.ShapeDtypeStruct((N_NODES, 3 * HID), _f32),
        ],
    )(x, wembt, bemb, wgt, bg, whht, bhh)


def _gru_mid_call(parts, h, gh, wiht, bih, wgt, bg, whht, bhh):
    return pl.pallas_call(
        _gru_mid_body,
        grid=(GRID,),
        in_specs=[
            _part_spec(0), _part_spec(1),
            _row_spec(HID), _row_spec(3 * HID),
            _full_spec(HID, 3 * HID), _full_spec(1, 3 * HID),
            _full_spec(HID, HID), _full_spec(1, HID),
            _full_spec(HID, 3 * HID), _full_spec(1, 3 * HID),
        ],
        out_specs=[_row_spec(HID), _row_spec(HID), _row_spec(3 * HID)],
        out_shape=[
            jax.ShapeDtypeStruct((N_NODES, HID), _f32),
            jax.ShapeDtypeStruct((N_NODES, HID), _f32),
            jax.ShapeDtypeStruct((N_NODES, 3 * HID), _f32),
        ],
    )(parts, parts, h, gh, wiht, bih, wgt, bg, whht, bhh)


def _gru_last_call(parts, h, gh, wiht, bih, woutt, bout):
    return pl.pallas_call(
        _gru_last_body,
        grid=(GRID,),
        in_specs=[
            _part_spec(0), _part_spec(1),
            _row_spec(HID), _row_spec(3 * HID),
            _full_spec(HID, 3 * HID), _full_spec(1, 3 * HID),
            _full_spec(HID, HID), _full_spec(1, HID),
        ],
        out_specs=_row_spec(HID),
        out_shape=jax.ShapeDtypeStruct((N_NODES, HID), _f32),
    )(parts, parts, h, gh, wiht, bih, woutt, bout)


@functools.partial(
    pl.kernel,
    out_type=jax.ShapeDtypeStruct((NC, N_PAD, HID), _f32),
    mesh=plsc.VectorSubcoreMesh(core_axis_name="c", subcore_axis_name="s"),
    scratch_types=[
        pltpu.VMEM((CHUNK,), jnp.int32),
        pltpu.VMEM((CHUNK,), jnp.int32),
        pltpu.VMEM((CHUNK, HID), _f32),
        pltpu.VMEM_SHARED((N_PAD, HID), _f32),
        pltpu.SemaphoreType.DMA,
    ],
)
def _msg_pass(m_hbm, src_hbm, dst_hbm, zeros_hbm, out_hbm,
              src_v, dst_v, rows_v, a_sh, sem):
    c = lax.axis_index("c")
    s = lax.axis_index("s")
    wid = c * NS + s

    pltpu.sync_copy(zeros_hbm.at[pl.ds(s * ROWS_PER_TILE, ROWS_PER_TILE)],
                    a_sh.at[pl.ds(s * ROWS_PER_TILE, ROWS_PER_TILE)])
    plsc.subcore_barrier()

    def do_chunk(chunk):
        off = chunk * CHUNK
        pltpu.sync_copy(src_hbm.at[pl.ds(off, CHUNK)], src_v)
        pltpu.sync_copy(dst_hbm.at[pl.ds(off, CHUNK)], dst_v)
        pltpu.async_copy(m_hbm.at[src_v], rows_v, sem).wait()
        pltpu.sync_copy(rows_v, a_sh.at[dst_v], add=True)

    @pl.loop(0, CHUNKS_FULL)
    def _(j):
        do_chunk(j * NW + wid)

    @pl.when(wid < CHUNKS_REM)
    def _():
        do_chunk(CHUNKS_FULL * NW + wid)

    plsc.subcore_barrier()
    pltpu.sync_copy(a_sh.at[pl.ds(s * ROWS_PER_TILE, ROWS_PER_TILE)],
                    out_hbm.at[c, pl.ds(s * ROWS_PER_TILE, ROWS_PER_TILE)])


def kernel(x, edge_index, W_emb, b_emb, W_g, b_g, W_ih, W_hh, b_ih, b_hh,
           W_out, b_out):
    src = edge_index[0]
    dst = edge_index[1]
    zeros = jnp.zeros((N_PAD, HID), _f32)

    wembt = W_emb.T
    wgt = W_g.T
    wiht = W_ih.T
    whht = W_hh.T
    woutt = W_out.T
    bemb = b_emb.reshape(1, HID)
    bg = b_g.reshape(1, HID)
    bih = b_ih.reshape(1, 3 * HID)
    bhh = b_hh.reshape(1, 3 * HID)
    bout = b_out.reshape(1, HID)

    h, m, gh = _embed_call(x, wembt, bemb, wgt, bg, whht, bhh)
    for step in range(N_STEPS):
        parts = _msg_pass(m, src, dst, zeros)
        if step < N_STEPS - 1:
            h, m, gh = _gru_mid_call(parts, h, gh, wiht, bih, wgt, bg,
                                     whht, bhh)
        else:
            return _gru_last_call(parts, h, gh, wiht, bih, woutt, bout)
